# Initial kernel scaffold; baseline (speedup 1.0000x reference)
import jax
import jax.numpy as jnp
from jax.experimental import pallas as pl


def kernel(x, table, w1, b1, w2, b2, w3, b3, w4, b4, ws, bs):
    raise NotImplementedError("write your pallas kernel here")


# speedup vs baseline: 2.5634x; 2.5634x over previous
import functools

import jax
import jax.numpy as jnp
from jax import lax
from jax.experimental import pallas as pl
from jax.experimental.pallas import tpu as pltpu
from jax.experimental.pallas import tpu_sc as plsc

VOCAB = 1000000
EMB = 16
HID = 32
BATCH = 16384

NC = 2
NS = 16
NW = NC * NS
NIDX = 4 * BATCH
B_PER_W = NIDX // NW
CHUNK = 128
NCHUNK = B_PER_W // CHUNK


def _make_gather():
    mesh = plsc.VectorSubcoreMesh(
        core_axis_name="c", subcore_axis_name="s", num_cores=NC, num_subcores=NS
    )

    @functools.partial(
        pl.kernel,
        mesh=mesh,
        compiler_params=pltpu.CompilerParams(use_tc_tiling_on_sc=False),
        out_type=jax.ShapeDtypeStruct((NIDX, EMB), jnp.float32),
        scratch_types=[
            pltpu.VMEM((NCHUNK, CHUNK), jnp.int32),
            pltpu.VMEM((B_PER_W, EMB), jnp.float32),
            pltpu.SemaphoreType.DMA,
        ],
    )
    def gather_kernel(idx_hbm, table_hbm, out_hbm, idx_v, rows_v, sem):
        wid = lax.axis_index("s") * NC + lax.axis_index("c")
        base = wid * B_PER_W
        pltpu.sync_copy(idx_hbm.at[wid], idx_v)
        copies = []
        for j in range(NCHUNK):
            copies.append(
                pltpu.async_copy(
                    table_hbm.at[idx_v.at[j]],
                    rows_v.at[pl.ds(j * CHUNK, CHUNK)],
                    sem,
                )
            )
        for c in copies:
            c.wait()
        pltpu.sync_copy(rows_v, out_hbm.at[pl.ds(base, B_PER_W)])

    return gather_kernel


_gather = _make_gather()

BB = 2048
GRID = BATCH // BB


def _mlp_body(e_ref, w_ref, b_ref, ws_ref, bs_ref, o_ref):
    acc = jnp.zeros((BB, HID), jnp.float32)
    for i in range(4):
        h = jnp.dot(e_ref[i], w_ref[i], preferred_element_type=jnp.float32) + b_ref[i]
        acc = acc + jnp.maximum(h, 0.0)
    o_ref[...] = (
        jnp.dot(acc, ws_ref[...], preferred_element_type=jnp.float32) + bs_ref[...]
    )


def kernel(x, table, w1, b1, w2, b2, w3, b3, w4, b4, ws, bs):
    idx = jnp.stack([x[:, 0], x[:, 1], x[:, 3], x[:, 4]], axis=0)
    idx = idx.reshape(NW, NCHUNK, CHUNK)

    rows = _gather(idx, table)
    e = rows.reshape(4, BATCH, EMB)

    w_all = jnp.stack([w1, w2, w3, w4], axis=0)
    b_all = jnp.stack([b1, b2, b3, b4], axis=0)[:, None, :]

    out = pl.pallas_call(
        _mlp_body,
        grid=(GRID,),
        in_specs=[
            pl.BlockSpec((4, BB, EMB), lambda i: (0, i, 0)),
            pl.BlockSpec((4, EMB, HID), lambda i: (0, 0, 0)),
            pl.BlockSpec((4, 1, HID), lambda i: (0, 0, 0)),
            pl.BlockSpec((HID, EMB), lambda i: (0, 0)),
            pl.BlockSpec((1, EMB), lambda i: (0, 0)),
        ],
        out_specs=pl.BlockSpec((BB, EMB), lambda i: (i, 0)),
        out_shape=jax.ShapeDtypeStruct((BATCH, EMB), jnp.float32),
    )(e, w_all, b_all, ws, bs[None, :])
    return out
